# Initial kernel scaffold; baseline (speedup 1.0000x reference)
import jax
import jax.numpy as jnp
from jax.experimental import pallas as pl


def kernel(element_embeddings, element_to_sample_map, num_samples, Wq, Wk, Wo):
    raise NotImplementedError("write your pallas kernel here")


# speedup vs baseline: 10.7182x; 10.7182x over previous
import jax
import jax.numpy as jnp
from jax.experimental import pallas as pl
from jax.experimental.pallas import tpu as pltpu

TOTAL = 32768
D = 512
S = 16
BLK = 1024
NBLK = TOTAL // BLK
NEG = -1e30


def _qk_kernel(x_ref, seg_ref, wq_ref, wk_ref, qk_ref, acc_ref, cnt_ref):
    i = pl.program_id(0)

    @pl.when(i == 0)
    def _():
        acc_ref[...] = jnp.zeros_like(acc_ref)
        cnt_ref[...] = jnp.zeros_like(cnt_ref)

    seg = seg_ref[0, 0, :]
    oh = (seg[:, None] == jax.lax.broadcasted_iota(jnp.int32, (BLK, S), 1)).astype(
        jnp.float32
    )
    x = x_ref[...]
    acc_ref[...] += jax.lax.dot_general(oh, x, (((0,), (0,)), ((), ())))
    cnt_ref[...] += jnp.sum(oh, axis=0, keepdims=True)

    @pl.when(i == NBLK - 1)
    def _():
        cnt = jnp.maximum(cnt_ref[0, :], 1.0)
        mean = acc_ref[...] / cnt[:, None]
        q = jnp.dot(mean, wq_ref[...])
        qk_ref[...] = jax.lax.dot_general(q, wk_ref[...], (((1,), (1,)), ((), ())))


def _attn_kernel(x_ref, seg_ref, qk_ref, wo_ref, out_ref, m_ref, d_ref, z_ref):
    i = pl.program_id(0)

    @pl.when(i == 0)
    def _():
        m_ref[...] = jnp.full_like(m_ref, NEG)
        d_ref[...] = jnp.zeros_like(d_ref)
        z_ref[...] = jnp.zeros_like(z_ref)

    seg = seg_ref[0, 0, :]
    ohb = seg[:, None] == jax.lax.broadcasted_iota(jnp.int32, (BLK, S), 1)
    oh = ohb.astype(jnp.float32)
    x = x_ref[...]
    qk_rows = jnp.dot(oh, qk_ref[...])
    scores = jnp.sum(x * qk_rows, axis=1)
    bm = jnp.max(jnp.where(ohb, scores[:, None], NEG), axis=0)
    m_old = m_ref[0, :]
    m_new = jnp.maximum(m_old, bm)
    scale = jnp.exp(m_old - m_new)
    m_row = jnp.dot(oh, m_new[:, None])
    w = jnp.exp(scores[:, None] - m_row)
    W = oh * w
    d_ref[0, :] = d_ref[0, :] * scale + jnp.sum(W, axis=0)
    z_ref[...] = z_ref[...] * scale[:, None] + jax.lax.dot_general(
        W, x, (((0,), (0,)), ((), ()))
    )
    m_ref[0, :] = m_new

    @pl.when(i == NBLK - 1)
    def _():
        d = d_ref[0, :]
        dd = jnp.where(d > 0, d, 1.0)
        out_ref[...] = jnp.dot(z_ref[...] / dd[:, None], wo_ref[...])


def kernel(element_embeddings, element_to_sample_map, num_samples, Wq, Wk, Wo):
    x = element_embeddings
    seg3 = element_to_sample_map.astype(jnp.int32).reshape(NBLK, 1, BLK)

    qk = pl.pallas_call(
        _qk_kernel,
        grid=(NBLK,),
        in_specs=[
            pl.BlockSpec((BLK, D), lambda i: (i, 0)),
            pl.BlockSpec((1, 1, BLK), lambda i: (i, 0, 0)),
            pl.BlockSpec((D, D), lambda i: (0, 0)),
            pl.BlockSpec((D, D), lambda i: (0, 0)),
        ],
        out_specs=pl.BlockSpec((S, D), lambda i: (0, 0)),
        out_shape=jax.ShapeDtypeStruct((S, D), jnp.float32),
        scratch_shapes=[
            pltpu.VMEM((S, D), jnp.float32),
            pltpu.VMEM((1, S), jnp.float32),
        ],
    )(x, seg3, Wq, Wk)

    out = pl.pallas_call(
        _attn_kernel,
        grid=(NBLK,),
        in_specs=[
            pl.BlockSpec((BLK, D), lambda i: (i, 0)),
            pl.BlockSpec((1, 1, BLK), lambda i: (i, 0, 0)),
            pl.BlockSpec((S, D), lambda i: (0, 0)),
            pl.BlockSpec((D, D), lambda i: (0, 0)),
        ],
        out_specs=pl.BlockSpec((S, D), lambda i: (0, 0)),
        out_shape=jax.ShapeDtypeStruct((S, D), jnp.float32),
        scratch_shapes=[
            pltpu.VMEM((1, S), jnp.float32),
            pltpu.VMEM((1, S), jnp.float32),
            pltpu.VMEM((S, D), jnp.float32),
        ],
    )(x, seg3, qk, Wo)
    return out
